# baseline (speedup 1.0000x reference)
import functools

import jax
import jax.numpy as jnp
from jax import lax
from jax.experimental import pallas as pl
from jax.experimental.pallas import tpu as pltpu
from jax.experimental.pallas import tpu_sc as plsc

B = 16384
DIM = 64
TEXT_DIM = 768

NC = 2
NS = 16
NW = NC * NS

SLABS = (8192, 8192)
CH = 64


def _sc_gather(sb, s_off, Q, prompt, P_pad, model, cat_pad, category):
    bpw = sb // NW
    nchunk = bpw // CH
    mesh = plsc.VectorSubcoreMesh(core_axis_name="c", subcore_axis_name="s")

    @functools.partial(
        pl.kernel,
        mesh=mesh,
        out_type=[
            jax.ShapeDtypeStruct((sb // 2, 2, TEXT_DIM), jnp.bfloat16),
            jax.ShapeDtypeStruct((sb, 2 * DIM), jnp.float32),
        ],
        scratch_types=[
            pltpu.VMEM((CH,), jnp.int32),
            pltpu.VMEM((CH,), jnp.int32),
            pltpu.VMEM((CH, TEXT_DIM), jnp.float32),
            pltpu.VMEM((CH, TEXT_DIM), jnp.float32),
            pltpu.VMEM((CH,), jnp.int32),
            pltpu.VMEM((CH,), jnp.int32),
            pltpu.VMEM((CH, 2 * DIM), jnp.float32),
            pltpu.VMEM((CH, 2 * DIM), jnp.float32),
            pltpu.VMEM((CH, 2 * DIM), jnp.float32),
            pltpu.VMEM((8, 2, TEXT_DIM), jnp.bfloat16),
            pltpu.SemaphoreType.DMA,
            pltpu.SemaphoreType.DMA,
            pltpu.SemaphoreType.DMA,
            pltpu.SemaphoreType.DMA,
        ],
    )
    def k(q_hbm, prompt_hbm, ptab_hbm, model_hbm, ctab_hbm, cat_hbm,
          qout_hbm, stout_hbm,
          qidx0, qidx1, qrows0, qrows1, pidx, vidx, prow, vrow, strow, qb16,
          qsem0, qsem1, psem, vsem):
        wid = lax.axis_index("s") * NC + lax.axis_index("c")
        base = s_off + wid * bpw
        obase = wid * bpw
        qidx = (qidx0, qidx1)
        qrows = (qrows0, qrows1)
        qsems = (qsem0, qsem1)

        pltpu.sync_copy(prompt_hbm.at[pl.ds(base, CH)], qidx0)
        qh = pltpu.async_copy(q_hbm.at[qidx0], qrows0, qsem0)
        pltpu.sync_copy(model_hbm.at[pl.ds(base, CH)], pidx)
        ph = pltpu.async_copy(ptab_hbm.at[pidx], prow, psem)
        pltpu.sync_copy(cat_hbm.at[pl.ds(base, CH)], vidx)
        vh = pltpu.async_copy(ctab_hbm.at[vidx], vrow, vsem)

        for j in range(nchunk):
            cur = j % 2
            nxt = (j + 1) % 2
            off = base + j * CH
            oout = obase + j * CH
            if j + 1 < nchunk:
                pltpu.sync_copy(prompt_hbm.at[pl.ds(off + CH, CH)], qidx[nxt])
                qh_next = pltpu.async_copy(q_hbm.at[qidx[nxt]], qrows[nxt],
                                           qsems[nxt])
            qh.wait()
            for hh in range(CH // 16):

                @pl.loop(0, 8)
                def _(r):
                    for kk in range(TEXT_DIM // 16):
                        c = kk * 16
                        for par in range(2):
                            x = qrows[cur].at[pl.ds(hh * 16 + 2 * r + par, 1),
                                              pl.ds(c, 16)][...]
                            qb16.at[pl.ds(r, 1), par, pl.ds(c, 16)][...] = (
                                x.astype(jnp.bfloat16))

                pltpu.sync_copy(
                    qb16, qout_hbm.at[pl.ds((oout + hh * 16) // 2, 8)])
            ph.wait()
            vh.wait()

            @pl.loop(0, CH)
            def _(r):
                for kk in range(DIM // 16):
                    c = kk * 16
                    p16 = prow.at[pl.ds(r, 1), pl.ds(c, 16)][...]
                    v16 = vrow.at[pl.ds(r, 1), pl.ds(c, 16)][...]
                    strow.at[pl.ds(r, 1), pl.ds(c, 16)][...] = p16 + v16
                    strow.at[pl.ds(r, 1), pl.ds(DIM + c, 16)][...] = p16 * v16

            pltpu.sync_copy(strow, stout_hbm.at[pl.ds(oout, CH)])
            if j + 1 < nchunk:
                pltpu.sync_copy(model_hbm.at[pl.ds(off + CH, CH)], pidx)
                ph = pltpu.async_copy(ptab_hbm.at[pidx], prow, psem)
                pltpu.sync_copy(cat_hbm.at[pl.ds(off + CH, CH)], vidx)
                vh = pltpu.async_copy(ctab_hbm.at[vidx], vrow, vsem)
                qh = qh_next

    return k(Q, prompt, P_pad, model, cat_pad, category)


def _tc_body(qraw_ref, st_ref, wtT_ref, bt_ref, wcT_ref, bcT_ref,
             out_ref):
    qr = qraw_ref[...].reshape(qraw_ref.shape[0] * 2, TEXT_DIM)
    q = lax.dot_general(qr, wtT_ref[...].astype(jnp.bfloat16),
                        (((1,), (1,)), ((), ())),
                        preferred_element_type=jnp.float32) + bt_ref[...]
    h = q * st_ref[:, :DIM] + st_ref[:, DIM:]
    out_ref[...] = lax.dot_general(wcT_ref[...], h,
                                   (((1,), (1,)), ((), ())),
                                   preferred_element_type=jnp.float32
                                   ) + bcT_ref[...]


def _tc_fm(sb, q_raw, st, wtT, bt2, wcT, bcT):
    TB = 2048
    return pl.pallas_call(
        _tc_body,
        grid=(sb // TB,),
        in_specs=[
            pl.BlockSpec((TB // 2, 2, TEXT_DIM), lambda i: (i, 0, 0)),
            pl.BlockSpec((TB, 2 * DIM), lambda i: (i, 0)),
            pl.BlockSpec((DIM, TEXT_DIM), lambda i: (0, 0)),
            pl.BlockSpec((1, DIM), lambda i: (0, 0)),
            pl.BlockSpec((2, DIM), lambda i: (0, 0)),
            pl.BlockSpec((2, 1), lambda i: (0, 0)),
        ],
        out_specs=pl.BlockSpec((2, TB), lambda i: (0, i)),
        out_shape=jax.ShapeDtypeStruct((2, sb), jnp.float32),
    )(q_raw, st, wtT, bt2, wcT, bcT)


def kernel(model, prompt, category, P, Q, W_text, b_text, cat_emb, W_cls, b_cls):
    P_pad = jnp.pad(P, ((0, 0), (0, DIM)))
    cat_pad = jnp.pad(cat_emb, ((0, 0), (0, DIM)))
    bt2 = b_text.reshape(1, DIM)
    wtT = W_text.T
    wcT = W_cls.T
    bcT = b_cls.reshape(2, 1)

    outs = []
    off = 0
    for sb in SLABS:
        q_raw, st = _sc_gather(sb, off, Q, prompt, P_pad, model, cat_pad,
                               category)
        off += sb
        outs.append(_tc_fm(sb, q_raw, st, wtT, bt2, wcT, bcT))
    return jnp.concatenate(outs, axis=1).T


# speedup vs baseline: 2.1484x; 2.1484x over previous
import functools

import jax
import jax.numpy as jnp
from jax import lax
from jax.experimental import pallas as pl
from jax.experimental.pallas import tpu as pltpu
from jax.experimental.pallas import tpu_sc as plsc

B = 16384
DIM = 64
TEXT_DIM = 768

NC = 2
NS = 16
NW = NC * NS

SLABS = (8192, 8192)
CH = 64


def _sc_gather(sb, s_off, Q, prompt, P_pad, model, cat_pad, category):
    bpw = sb // NW
    nchunk = bpw // CH
    mesh = plsc.VectorSubcoreMesh(core_axis_name="c", subcore_axis_name="s")

    @functools.partial(
        pl.kernel,
        mesh=mesh,
        out_type=[
            jax.ShapeDtypeStruct((sb, TEXT_DIM), jnp.float32),
            jax.ShapeDtypeStruct((sb, 2 * DIM), jnp.float32),
        ],
        scratch_types=[
            pltpu.VMEM((CH,), jnp.int32),
            pltpu.VMEM((CH,), jnp.int32),
            pltpu.VMEM((CH, TEXT_DIM), jnp.float32),
            pltpu.VMEM((CH, TEXT_DIM), jnp.float32),
            pltpu.VMEM((CH,), jnp.int32),
            pltpu.VMEM((CH,), jnp.int32),
            pltpu.VMEM((CH, 2 * DIM), jnp.float32),
            pltpu.VMEM((CH, 2 * DIM), jnp.float32),
            pltpu.VMEM((CH, 2 * DIM), jnp.float32),
            pltpu.SemaphoreType.DMA,
            pltpu.SemaphoreType.DMA,
            pltpu.SemaphoreType.DMA,
            pltpu.SemaphoreType.DMA,
        ],
    )
    def k(q_hbm, prompt_hbm, ptab_hbm, model_hbm, ctab_hbm, cat_hbm,
          qout_hbm, stout_hbm,
          qidx0, qidx1, qrows0, qrows1, pidx, vidx, prow, vrow, strow,
          qsem0, qsem1, psem, vsem):
        wid = lax.axis_index("s") * NC + lax.axis_index("c")
        base = s_off + wid * bpw
        obase = wid * bpw
        qidx = (qidx0, qidx1)
        qrows = (qrows0, qrows1)
        qsems = (qsem0, qsem1)

        pltpu.sync_copy(prompt_hbm.at[pl.ds(base, CH)], qidx0)
        qh = pltpu.async_copy(q_hbm.at[qidx0], qrows0, qsem0)
        pltpu.sync_copy(model_hbm.at[pl.ds(base, CH)], pidx)
        ph = pltpu.async_copy(ptab_hbm.at[pidx], prow, psem)
        pltpu.sync_copy(cat_hbm.at[pl.ds(base, CH)], vidx)
        vh = pltpu.async_copy(ctab_hbm.at[vidx], vrow, vsem)

        for j in range(nchunk):
            cur = j % 2
            nxt = (j + 1) % 2
            off = base + j * CH
            oout = obase + j * CH
            if j + 1 < nchunk:
                pltpu.sync_copy(prompt_hbm.at[pl.ds(off + CH, CH)], qidx[nxt])
                qh_next = pltpu.async_copy(q_hbm.at[qidx[nxt]], qrows[nxt],
                                           qsems[nxt])
            qh.wait()
            pltpu.sync_copy(qrows[cur], qout_hbm.at[pl.ds(oout, CH)])
            ph.wait()
            vh.wait()

            @pl.loop(0, CH)
            def _(r):
                for kk in range(DIM // 16):
                    c = kk * 16
                    p16 = prow.at[pl.ds(r, 1), pl.ds(c, 16)][...]
                    v16 = vrow.at[pl.ds(r, 1), pl.ds(c, 16)][...]
                    strow.at[pl.ds(r, 1), pl.ds(c, 16)][...] = p16 + v16
                    strow.at[pl.ds(r, 1), pl.ds(DIM + c, 16)][...] = p16 * v16

            pltpu.sync_copy(strow, stout_hbm.at[pl.ds(oout, CH)])
            if j + 1 < nchunk:
                pltpu.sync_copy(model_hbm.at[pl.ds(off + CH, CH)], pidx)
                ph = pltpu.async_copy(ptab_hbm.at[pidx], prow, psem)
                pltpu.sync_copy(cat_hbm.at[pl.ds(off + CH, CH)], vidx)
                vh = pltpu.async_copy(ctab_hbm.at[vidx], vrow, vsem)
                qh = qh_next

    return k(Q, prompt, P_pad, model, cat_pad, category)


def _tc_body(qraw_ref, st_ref, wtT_ref, bt_ref, wcT_ref, bcT_ref,
             out_ref):
    q = lax.dot_general(qraw_ref[...], wtT_ref[...],
                        (((1,), (1,)), ((), ())),
                        preferred_element_type=jnp.float32) + bt_ref[...]
    h = q * st_ref[:, :DIM] + st_ref[:, DIM:]
    out_ref[...] = lax.dot_general(wcT_ref[...], h,
                                   (((1,), (1,)), ((), ())),
                                   preferred_element_type=jnp.float32
                                   ) + bcT_ref[...]


def _tc_fm(sb, q_raw, st, wtT, bt2, wcT, bcT):
    TB = 2048
    return pl.pallas_call(
        _tc_body,
        grid=(sb // TB,),
        in_specs=[
            pl.BlockSpec((TB, TEXT_DIM), lambda i: (i, 0)),
            pl.BlockSpec((TB, 2 * DIM), lambda i: (i, 0)),
            pl.BlockSpec((DIM, TEXT_DIM), lambda i: (0, 0)),
            pl.BlockSpec((1, DIM), lambda i: (0, 0)),
            pl.BlockSpec((2, DIM), lambda i: (0, 0)),
            pl.BlockSpec((2, 1), lambda i: (0, 0)),
        ],
        out_specs=pl.BlockSpec((2, TB), lambda i: (0, i)),
        out_shape=jax.ShapeDtypeStruct((2, sb), jnp.float32),
    )(q_raw, st, wtT, bt2, wcT, bcT)


def kernel(model, prompt, category, P, Q, W_text, b_text, cat_emb, W_cls, b_cls):
    P_pad = jnp.pad(P, ((0, 0), (0, DIM)))
    cat_pad = jnp.pad(cat_emb, ((0, 0), (0, DIM)))
    bt2 = b_text.reshape(1, DIM)
    wtT = W_text.T
    wcT = W_cls.T
    bcT = b_cls.reshape(2, 1)

    outs = []
    off = 0
    for sb in SLABS:
        q_raw, st = _sc_gather(sb, off, Q, prompt, P_pad, model, cat_pad,
                               category)
        off += sb
        outs.append(_tc_fm(sb, q_raw, st, wtT, bt2, wcT, bcT))
    return jnp.concatenate(outs, axis=1).T
